# Initial kernel scaffold; baseline (speedup 1.0000x reference)
import functools

import jax
import jax.numpy as jnp
from jax import lax
from jax.experimental import pallas as pl
from jax.experimental.pallas import tpu as pltpu
from jax.experimental.pallas import tpu_sc as plsc

N = 10000
E = 320000
D_IN = 128
D_HID = 256
D_OUT = 128

NC = 2
NS = 16
ROWS_PER_TILE = 640
NPAD = NS * ROWS_PER_TILE
CHUNK = 80
RB = 400


def _sc_mesh():
    return plsc.VectorSubcoreMesh(core_axis_name="c", subcore_axis_name="s")


@functools.partial(
    pl.kernel,
    mesh=_sc_mesh(),
    out_type=jax.ShapeDtypeStruct((NC, NPAD), jnp.float32),
    scratch_types=[
        pltpu.VMEM((CHUNK,), jnp.int32),
        pltpu.VMEM((CHUNK,), jnp.float32),
        pltpu.VMEM((ROWS_PER_TILE,), jnp.float32),
        pltpu.VMEM_SHARED((NPAD,), jnp.float32),
    ],
)
def _deg(dst_hbm, out_hbm, idx_v, ones_v, zb_v, acc_sh):
    c = lax.axis_index("c")
    s = lax.axis_index("s")
    for i in range(CHUNK // 16):
        ones_v[pl.ds(i * 16, 16)] = jnp.full((16,), 1.0, jnp.float32)

    def zfill(i, carry):
        zb_v[pl.ds(i * 16, 16)] = jnp.zeros((16,), jnp.float32)
        return carry

    lax.fori_loop(0, ROWS_PER_TILE // 16, zfill, 0)
    row0 = s * ROWS_PER_TILE
    pltpu.sync_copy(zb_v, acc_sh.at[pl.ds(row0, ROWS_PER_TILE)])
    plsc.subcore_barrier()

    epw = E // (NC * NS)
    base = (c * NS + s) * epw

    def step(i, carry):
        pltpu.sync_copy(dst_hbm.at[pl.ds(base + i * CHUNK, CHUNK)], idx_v)
        pltpu.sync_copy(ones_v, acc_sh.at[idx_v], add=True)
        return carry

    lax.fori_loop(0, epw // CHUNK, step, 0)
    plsc.subcore_barrier()

    pltpu.sync_copy(acc_sh.at[pl.ds(row0, ROWS_PER_TILE)], zb_v)

    @pl.when(c == 0)
    def _():
        pltpu.sync_copy(zb_v, out_hbm.at[0, pl.ds(row0, ROWS_PER_TILE)])

    @pl.when(c == 1)
    def _():
        pltpu.sync_copy(zb_v, out_hbm.at[1, pl.ds(row0, ROWS_PER_TILE)])


def _make_conv(F2):
    ept = E // NS
    nchunk = ept // CHUNK
    zr = 64
    nz = ROWS_PER_TILE // zr

    @functools.partial(
        pl.kernel,
        mesh=_sc_mesh(),
        out_type=[
            jax.ShapeDtypeStruct((NPAD, F2), jnp.float32),
            jax.ShapeDtypeStruct((NPAD, F2), jnp.float32),
        ],
        scratch_types=[
            pltpu.VMEM((CHUNK,), jnp.int32),
            pltpu.VMEM((CHUNK,), jnp.int32),
            pltpu.VMEM((CHUNK, F2), jnp.float32),
            pltpu.VMEM((zr, F2), jnp.float32),
            pltpu.SemaphoreType.DMA,
            pltpu.VMEM_SHARED((NPAD, F2), jnp.float32),
        ],
    )
    def conv(src_hbm, dst_hbm, hs0_hbm, hs1_hbm, out0_hbm, out1_hbm,
             si_v, di_v, rows_v, zb_v, sem, acc_sh):
        c = lax.axis_index("c")
        s = lax.axis_index("s")

        def zrow(j, carry):
            def zcol(l, carry2):
                zb_v[j, pl.ds(l * 16, 16)] = jnp.zeros((16,), jnp.float32)
                return carry2
            return lax.fori_loop(0, F2 // 16, zcol, carry)

        lax.fori_loop(0, zr, zrow, 0)
        row0 = s * ROWS_PER_TILE
        for t in range(nz):
            pltpu.sync_copy(zb_v, acc_sh.at[pl.ds(row0 + t * zr, zr)])
        plsc.subcore_barrier()

        def pass_edges(hs_hbm):
            base = s * ept

            def step(i, carry):
                off = base + i * CHUNK
                pltpu.sync_copy(src_hbm.at[pl.ds(off, CHUNK)], si_v)
                pltpu.sync_copy(dst_hbm.at[pl.ds(off, CHUNK)], di_v)
                pltpu.async_copy(hs_hbm.at[si_v], rows_v, sem).wait()
                pltpu.sync_copy(rows_v, acc_sh.at[di_v], add=True)
                return carry

            lax.fori_loop(0, nchunk, step, 0)

        @pl.when(c == 0)
        def _():
            pass_edges(hs0_hbm)

        @pl.when(c == 1)
        def _():
            pass_edges(hs1_hbm)

        plsc.subcore_barrier()
        for t in range(nz):
            pltpu.sync_copy(acc_sh.at[pl.ds(row0 + t * zr, zr)], zb_v)

            @pl.when(c == 0)
            def _():
                pltpu.sync_copy(zb_v, out0_hbm.at[pl.ds(row0 + t * zr, zr)])

            @pl.when(c == 1)
            def _():
                pltpu.sync_copy(zb_v, out1_hbm.at[pl.ds(row0 + t * zr, zr)])

    return conv


_conv_hid = _make_conv(D_HID // 2)
_conv_out = _make_conv(D_OUT // 2)


def _mm1_body(degp_ref, x_ref, w_ref, hs0_ref, hs1_ref, dinv_ref):
    deg = degp_ref[:, 0] + degp_ref[:, 1] + 1.0
    dinv = lax.rsqrt(deg)
    h = jnp.dot(x_ref[...], w_ref[...], preferred_element_type=jnp.float32)
    hs = h * dinv[:, None]
    hs0_ref[...] = hs[:, : D_HID // 2]
    hs1_ref[...] = hs[:, D_HID // 2:]
    dinv_ref[...] = dinv[:, None]


_mm1 = pl.pallas_call(
    _mm1_body,
    grid=(N // RB,),
    in_specs=[
        pl.BlockSpec((RB, 2), lambda i: (i, 0)),
        pl.BlockSpec((RB, D_IN), lambda i: (i, 0)),
        pl.BlockSpec((D_IN, D_HID), lambda i: (0, 0)),
    ],
    out_specs=[
        pl.BlockSpec((RB, D_HID // 2), lambda i: (i, 0)),
        pl.BlockSpec((RB, D_HID // 2), lambda i: (i, 0)),
        pl.BlockSpec((RB, 1), lambda i: (i, 0)),
    ],
    out_shape=[
        jax.ShapeDtypeStruct((N, D_HID // 2), jnp.float32),
        jax.ShapeDtypeStruct((N, D_HID // 2), jnp.float32),
        jax.ShapeDtypeStruct((N, 1), jnp.float32),
    ],
)


def _mm2_body(acc0_ref, acc1_ref, hs0_ref, hs1_ref, dinv_ref, b1_ref, w2_ref,
              o0_ref, o1_ref):
    dinv = dinv_ref[...]
    pre = jnp.concatenate(
        [acc0_ref[...] + hs0_ref[...], acc1_ref[...] + hs1_ref[...]], axis=1)
    act = jnp.maximum(pre * dinv + b1_ref[...], 0.0)
    h2 = jnp.dot(act, w2_ref[...], preferred_element_type=jnp.float32) * dinv
    o0_ref[...] = h2[:, : D_OUT // 2]
    o1_ref[...] = h2[:, D_OUT // 2:]


_mm2 = pl.pallas_call(
    _mm2_body,
    grid=(N // RB,),
    in_specs=[
        pl.BlockSpec((RB, D_HID // 2), lambda i: (i, 0)),
        pl.BlockSpec((RB, D_HID // 2), lambda i: (i, 0)),
        pl.BlockSpec((RB, D_HID // 2), lambda i: (i, 0)),
        pl.BlockSpec((RB, D_HID // 2), lambda i: (i, 0)),
        pl.BlockSpec((RB, 1), lambda i: (i, 0)),
        pl.BlockSpec((1, D_HID), lambda i: (0, 0)),
        pl.BlockSpec((D_HID, D_OUT), lambda i: (0, 0)),
    ],
    out_specs=[
        pl.BlockSpec((RB, D_OUT // 2), lambda i: (i, 0)),
        pl.BlockSpec((RB, D_OUT // 2), lambda i: (i, 0)),
    ],
    out_shape=[
        jax.ShapeDtypeStruct((N, D_OUT // 2), jnp.float32),
        jax.ShapeDtypeStruct((N, D_OUT // 2), jnp.float32),
    ],
)


def _mm3_body(acc0_ref, acc1_ref, hs0_ref, hs1_ref, dinv_ref, b2_ref, out_ref):
    pre = jnp.concatenate(
        [acc0_ref[...] + hs0_ref[...], acc1_ref[...] + hs1_ref[...]], axis=1)
    out_ref[...] = pre * dinv_ref[...] + b2_ref[...]


_mm3 = pl.pallas_call(
    _mm3_body,
    grid=(N // RB,),
    in_specs=[
        pl.BlockSpec((RB, D_OUT // 2), lambda i: (i, 0)),
        pl.BlockSpec((RB, D_OUT // 2), lambda i: (i, 0)),
        pl.BlockSpec((RB, D_OUT // 2), lambda i: (i, 0)),
        pl.BlockSpec((RB, D_OUT // 2), lambda i: (i, 0)),
        pl.BlockSpec((RB, 1), lambda i: (i, 0)),
        pl.BlockSpec((1, D_OUT), lambda i: (0, 0)),
    ],
    out_specs=pl.BlockSpec((RB, D_OUT), lambda i: (i, 0)),
    out_shape=jax.ShapeDtypeStruct((N, D_OUT), jnp.float32),
)


def kernel(x, edge_index, conv1_weight, conv1_bias, conv2_weight, conv2_bias):
    src = edge_index[0]
    dst = edge_index[1]
    deg_parts = _deg(dst)
    degp = deg_parts[:, :N].T
    hs1a, hs1b, dinv = _mm1(degp, x, conv1_weight)
    acc1a, acc1b = _conv_hid(src, dst, hs1a, hs1b)
    hs2a, hs2b = _mm2(acc1a, acc1b, hs1a, hs1b, dinv,
                      conv1_bias.reshape(1, -1), conv2_weight)
    acc2a, acc2b = _conv_out(src, dst, hs2a, hs2b)
    return _mm3(acc2a, acc2b, hs2a, hs2b, dinv, conv2_bias.reshape(1, -1))


# speedup vs baseline: 11.1160x; 11.1160x over previous
import functools

import jax
import jax.numpy as jnp
from jax import lax
from jax.experimental import pallas as pl
from jax.experimental.pallas import tpu as pltpu
from jax.experimental.pallas import tpu_sc as plsc

N = 10000
E = 320000
D_IN = 128
D_HID = 256
D_OUT = 128

NC = 2
NS = 16
ROWS_PER_TILE = 640
NPAD = NS * ROWS_PER_TILE
CHUNK = 80
RB = 400


def _sc_mesh():
    return plsc.VectorSubcoreMesh(core_axis_name="c", subcore_axis_name="s")


HR = NPAD // 128
HRPT = 8
DEG_CHUNK = 2000


@functools.partial(
    pl.kernel,
    mesh=_sc_mesh(),
    compiler_params=pltpu.CompilerParams(needs_layout_passes=False),
    out_type=jax.ShapeDtypeStruct((NC, HR, 128), jnp.float32),
    scratch_types=[
        pltpu.VMEM((DEG_CHUNK,), jnp.int32),
        pltpu.VMEM((HR, 128), jnp.float32),
        pltpu.VMEM((HR,), jnp.int32),
        pltpu.VMEM((HRPT, 128), jnp.float32),
        pltpu.VMEM_SHARED((HR, 128), jnp.float32),
    ],
)
def _deg(dst_hbm, out_hbm, idx_v, hist_v, rowid_v, ob_v, acc_sh):
    c = lax.axis_index("c")
    s = lax.axis_index("s")

    def zrow(j, carry):
        def zcol(l, carry2):
            hist_v[j, pl.ds(l * 16, 16)] = jnp.zeros((16,), jnp.float32)
            return carry2
        return lax.fori_loop(0, 128 // 16, zcol, carry)

    lax.fori_loop(0, HR, zrow, 0)
    for j in range(HR // 16):
        rowid_v[pl.ds(j * 16, 16)] = (
            lax.iota(jnp.int32, 16) + jnp.full((16,), j * 16, jnp.int32))

    @pl.when(s == 0)
    def _():
        pltpu.sync_copy(hist_v, acc_sh)
    plsc.subcore_barrier()

    epw = E // (NC * NS)
    base = (c * NS + s) * epw

    def chunk_step(k, carry):
        pltpu.sync_copy(dst_hbm.at[pl.ds(base + k * DEG_CHUNK, DEG_CHUNK)],
                        idx_v)

        def vstep(j, carry2):
            v = idx_v[pl.ds(j * 16, 16)]
            skey, _ = plsc.sort_key_val(v, v)
            cnt, last = plsc.scan_count(skey)
            row = lax.shift_right_logical(skey, 7)
            col = jnp.bitwise_and(skey, 127)
            plsc.addupdate_scatter(hist_v, [row, col],
                                   cnt.astype(jnp.float32), mask=last)
            return carry2

        return lax.fori_loop(0, DEG_CHUNK // 16, vstep, carry)

    lax.fori_loop(0, epw // DEG_CHUNK, chunk_step, 0)
    pltpu.sync_copy(hist_v, acc_sh.at[rowid_v], add=True)
    plsc.subcore_barrier()

    @pl.when(s < HR // HRPT)
    def _():
        pltpu.sync_copy(acc_sh.at[pl.ds(s * HRPT, HRPT)], ob_v)

        @pl.when(c == 0)
        def _():
            pltpu.sync_copy(ob_v, out_hbm.at[0, pl.ds(s * HRPT, HRPT)])

        @pl.when(c == 1)
        def _():
            pltpu.sync_copy(ob_v, out_hbm.at[1, pl.ds(s * HRPT, HRPT)])


def _make_conv(F2):
    ept = E // NS
    nchunk = ept // CHUNK
    zr = 64
    nz = ROWS_PER_TILE // zr

    @functools.partial(
        pl.kernel,
        mesh=_sc_mesh(),
        out_type=[
            jax.ShapeDtypeStruct((NPAD, F2), jnp.float32),
            jax.ShapeDtypeStruct((NPAD, F2), jnp.float32),
        ],
        scratch_types=[
            pltpu.VMEM((CHUNK,), jnp.int32),
            pltpu.VMEM((CHUNK,), jnp.int32),
            pltpu.VMEM((CHUNK, F2), jnp.float32),
            pltpu.VMEM((zr, F2), jnp.float32),
            pltpu.SemaphoreType.DMA,
            pltpu.VMEM_SHARED((NPAD, F2), jnp.float32),
        ],
    )
    def conv(src_hbm, dst_hbm, hs0_hbm, hs1_hbm, out0_hbm, out1_hbm,
             si_v, di_v, rows_v, zb_v, sem, acc_sh):
        c = lax.axis_index("c")
        s = lax.axis_index("s")

        def zrow(j, carry):
            def zcol(l, carry2):
                zb_v[j, pl.ds(l * 16, 16)] = jnp.zeros((16,), jnp.float32)
                return carry2
            return lax.fori_loop(0, F2 // 16, zcol, carry)

        lax.fori_loop(0, zr, zrow, 0)
        row0 = s * ROWS_PER_TILE
        for t in range(nz):
            pltpu.sync_copy(zb_v, acc_sh.at[pl.ds(row0 + t * zr, zr)])
        plsc.subcore_barrier()

        def pass_edges(hs_hbm):
            base = s * ept

            def step(i, carry):
                off = base + i * CHUNK
                pltpu.sync_copy(src_hbm.at[pl.ds(off, CHUNK)], si_v)
                pltpu.sync_copy(dst_hbm.at[pl.ds(off, CHUNK)], di_v)
                pltpu.async_copy(hs_hbm.at[si_v], rows_v, sem).wait()
                pltpu.sync_copy(rows_v, acc_sh.at[di_v], add=True)
                return carry

            lax.fori_loop(0, nchunk, step, 0)

        @pl.when(c == 0)
        def _():
            pass_edges(hs0_hbm)

        @pl.when(c == 1)
        def _():
            pass_edges(hs1_hbm)

        plsc.subcore_barrier()
        for t in range(nz):
            pltpu.sync_copy(acc_sh.at[pl.ds(row0 + t * zr, zr)], zb_v)

            @pl.when(c == 0)
            def _():
                pltpu.sync_copy(zb_v, out0_hbm.at[pl.ds(row0 + t * zr, zr)])

            @pl.when(c == 1)
            def _():
                pltpu.sync_copy(zb_v, out1_hbm.at[pl.ds(row0 + t * zr, zr)])

    return conv


_conv_hid = _make_conv(D_HID // 2)


def _make_conv_es(F):
    ept = E // (NC * NS)
    nchunk = ept // CHUNK
    zr = 64
    nz = ROWS_PER_TILE // zr

    @functools.partial(
        pl.kernel,
        mesh=_sc_mesh(),
        out_type=[
            jax.ShapeDtypeStruct((NPAD, F), jnp.float32),
            jax.ShapeDtypeStruct((NPAD, F), jnp.float32),
        ],
        scratch_types=[
            pltpu.VMEM((CHUNK,), jnp.int32),
            pltpu.VMEM((CHUNK,), jnp.int32),
            pltpu.VMEM((CHUNK, F), jnp.float32),
            pltpu.VMEM((zr, F), jnp.float32),
            pltpu.SemaphoreType.DMA,
            pltpu.VMEM_SHARED((NPAD, F), jnp.float32),
        ],
    )
    def conv(src_hbm, dst_hbm, hs_hbm, out0_hbm, out1_hbm,
             si_v, di_v, rows_v, zb_v, sem, acc_sh):
        c = lax.axis_index("c")
        s = lax.axis_index("s")

        def zrow(j, carry):
            def zcol(l, carry2):
                zb_v[j, pl.ds(l * 16, 16)] = jnp.zeros((16,), jnp.float32)
                return carry2
            return lax.fori_loop(0, F // 16, zcol, carry)

        lax.fori_loop(0, zr, zrow, 0)
        row0 = s * ROWS_PER_TILE
        for t in range(nz):
            pltpu.sync_copy(zb_v, acc_sh.at[pl.ds(row0 + t * zr, zr)])
        plsc.subcore_barrier()

        base = (c * NS + s) * ept

        def step(i, carry):
            off = base + i * CHUNK
            pltpu.sync_copy(src_hbm.at[pl.ds(off, CHUNK)], si_v)
            pltpu.sync_copy(dst_hbm.at[pl.ds(off, CHUNK)], di_v)
            pltpu.async_copy(hs_hbm.at[si_v], rows_v, sem).wait()
            pltpu.sync_copy(rows_v, acc_sh.at[di_v], add=True)
            return carry

        lax.fori_loop(0, nchunk, step, 0)
        plsc.subcore_barrier()
        for t in range(nz):
            pltpu.sync_copy(acc_sh.at[pl.ds(row0 + t * zr, zr)], zb_v)

            @pl.when(c == 0)
            def _():
                pltpu.sync_copy(zb_v, out0_hbm.at[pl.ds(row0 + t * zr, zr)])

            @pl.when(c == 1)
            def _():
                pltpu.sync_copy(zb_v, out1_hbm.at[pl.ds(row0 + t * zr, zr)])

    return conv


_conv_out = _make_conv_es(D_OUT)


def _mm1_body(degp_ref, x_ref, w_ref, hs0_ref, hs1_ref, dinv_ref):
    deg = degp_ref[:, 0] + degp_ref[:, 1] + 1.0
    dinv = lax.rsqrt(deg)
    h = jnp.dot(x_ref[...], w_ref[...], preferred_element_type=jnp.float32)
    hs = h * dinv[:, None]
    hs0_ref[...] = hs[:, : D_HID // 2]
    hs1_ref[...] = hs[:, D_HID // 2:]
    dinv_ref[...] = dinv[:, None]


_mm1 = pl.pallas_call(
    _mm1_body,
    grid=(N // RB,),
    in_specs=[
        pl.BlockSpec((RB, 2), lambda i: (i, 0)),
        pl.BlockSpec((RB, D_IN), lambda i: (i, 0)),
        pl.BlockSpec((D_IN, D_HID), lambda i: (0, 0)),
    ],
    out_specs=[
        pl.BlockSpec((RB, D_HID // 2), lambda i: (i, 0)),
        pl.BlockSpec((RB, D_HID // 2), lambda i: (i, 0)),
        pl.BlockSpec((RB, 1), lambda i: (i, 0)),
    ],
    out_shape=[
        jax.ShapeDtypeStruct((N, D_HID // 2), jnp.float32),
        jax.ShapeDtypeStruct((N, D_HID // 2), jnp.float32),
        jax.ShapeDtypeStruct((N, 1), jnp.float32),
    ],
)


def _mm2_body(acc0_ref, acc1_ref, hs0_ref, hs1_ref, dinv_ref, b1_ref, w2_ref,
              o_ref):
    dinv = dinv_ref[...]
    pre = jnp.concatenate(
        [acc0_ref[...] + hs0_ref[...], acc1_ref[...] + hs1_ref[...]], axis=1)
    act = jnp.maximum(pre * dinv + b1_ref[...], 0.0)
    o_ref[...] = jnp.dot(act, w2_ref[...],
                         preferred_element_type=jnp.float32) * dinv


_mm2 = pl.pallas_call(
    _mm2_body,
    grid=(N // RB,),
    in_specs=[
        pl.BlockSpec((RB, D_HID // 2), lambda i: (i, 0)),
        pl.BlockSpec((RB, D_HID // 2), lambda i: (i, 0)),
        pl.BlockSpec((RB, D_HID // 2), lambda i: (i, 0)),
        pl.BlockSpec((RB, D_HID // 2), lambda i: (i, 0)),
        pl.BlockSpec((RB, 1), lambda i: (i, 0)),
        pl.BlockSpec((1, D_HID), lambda i: (0, 0)),
        pl.BlockSpec((D_HID, D_OUT), lambda i: (0, 0)),
    ],
    out_specs=pl.BlockSpec((RB, D_OUT), lambda i: (i, 0)),
    out_shape=jax.ShapeDtypeStruct((N, D_OUT), jnp.float32),
)


def _mm3_body(acc0_ref, acc1_ref, hs2_ref, dinv_ref, b2_ref, out_ref):
    pre = acc0_ref[...] + acc1_ref[...] + hs2_ref[...]
    out_ref[...] = pre * dinv_ref[...] + b2_ref[...]


_mm3 = pl.pallas_call(
    _mm3_body,
    grid=(N // RB,),
    in_specs=[
        pl.BlockSpec((RB, D_OUT), lambda i: (i, 0)),
        pl.BlockSpec((RB, D_OUT), lambda i: (i, 0)),
        pl.BlockSpec((RB, D_OUT), lambda i: (i, 0)),
        pl.BlockSpec((RB, 1), lambda i: (i, 0)),
        pl.BlockSpec((1, D_OUT), lambda i: (0, 0)),
    ],
    out_specs=pl.BlockSpec((RB, D_OUT), lambda i: (i, 0)),
    out_shape=jax.ShapeDtypeStruct((N, D_OUT), jnp.float32),
)


def kernel(x, edge_index, conv1_weight, conv1_bias, conv2_weight, conv2_bias):
    src = edge_index[0]
    dst = edge_index[1]
    deg_parts = _deg(dst)
    degp = deg_parts.reshape(NC, NPAD)[:, :N].T
    hs1a, hs1b, dinv = _mm1(degp, x, conv1_weight)
    acc1a, acc1b = _conv_hid(src, dst, hs1a, hs1b)
    hs2 = _mm2(acc1a, acc1b, hs1a, hs1b, dinv,
               conv1_bias.reshape(1, -1), conv2_weight)
    acc2a, acc2b = _conv_out(src, dst, hs2)
    return _mm3(acc2a, acc2b, hs2, dinv, conv2_bias.reshape(1, -1))
